# baseline (speedup 1.0000x reference)
import functools

import jax
import jax.numpy as jnp
from jax import lax
from jax.experimental import pallas as pl
from jax.experimental.pallas import tpu as pltpu
from jax.experimental.pallas import tpu_sc as plsc

NUM_CORES = 2
NUM_SUBCORES = 16
NUM_WORKERS = NUM_CORES * NUM_SUBCORES
LANES = 16
NBUF = 4
SUBCHUNKS = 4


def _sc_body(x_hbm, p_hbm, o_hbm, p_v, xbuf, obuf, lsem, ssem):
    B = x_hbm.shape[0]
    flat = p_v.shape[0]
    chunk = flat // SUBCHUNKS
    nvec = chunk // LANES
    wid = lax.axis_index("s") * NUM_CORES + lax.axis_index("c")
    base = wid * flat

    nchunks = B * SUBCHUNKS

    def load(g, k):
        b, s = g // SUBCHUNKS, g % SUBCHUNKS
        return pltpu.make_async_copy(
            x_hbm.at[b, pl.ds(base + s * chunk, chunk)], xbuf.at[k], lsem.at[k])

    def store(g, k):
        b, s = g // SUBCHUNKS, g % SUBCHUNKS
        return pltpu.make_async_copy(
            obuf.at[k], o_hbm.at[b, pl.ds(base + s * chunk, chunk)], ssem.at[k])

    for k in range(NBUF):
        load(k, k).start()

    pltpu.sync_copy(p_hbm.at[pl.ds(base, flat)], p_v)

    for g in range(nchunks):
        k = g % NBUF
        off = (g % SUBCHUNKS) * chunk
        load(g, k).wait()
        if g >= NBUF:
            store(g - NBUF, k).wait()

        def add_one(i, _, k=k, off=off):
            sl = pl.ds(i * LANES, LANES)
            obuf[k, sl] = xbuf[k, sl] + p_v[pl.ds(off + i * LANES, LANES)]
            return _

        lax.fori_loop(0, nvec, add_one, None, unroll=8)

        store(g, k).start()
        if g + NBUF < nchunks:
            load(g + NBUF, k).start()

    for g in range(nchunks - NBUF, nchunks):
        store(g, g % NBUF).wait()


def kernel(x, pos_table):
    B, T, D = x.shape
    flat = T * D // NUM_WORKERS
    mesh = plsc.VectorSubcoreMesh(core_axis_name="c", subcore_axis_name="s")
    run = functools.partial(
        pl.kernel,
        mesh=mesh,
        out_type=jax.ShapeDtypeStruct((B, T * D), jnp.float32),
        scratch_types=[
            pltpu.VMEM((flat,), jnp.float32),
            pltpu.VMEM((NBUF, flat // SUBCHUNKS), jnp.float32),
            pltpu.VMEM((NBUF, flat // SUBCHUNKS), jnp.float32),
            pltpu.SemaphoreType.DMA((NBUF,)),
            pltpu.SemaphoreType.DMA((NBUF,)),
        ],
    )(_sc_body)
    out = run(x.reshape(B, T * D), pos_table.reshape(T * D))
    return out.reshape(B, T, D)


# speedup vs baseline: 1.4519x; 1.4519x over previous
import functools

import jax
import jax.numpy as jnp
from jax import lax
from jax.experimental import pallas as pl
from jax.experimental.pallas import tpu as pltpu
from jax.experimental.pallas import tpu_sc as plsc

NUM_CORES = 2
NUM_SUBCORES = 16
NUM_WORKERS = NUM_CORES * NUM_SUBCORES
LANES = 16
NBUF = 4
SUBCHUNKS = 4


def _sc_body(x_hbm, p_hbm, o_hbm, p_v, xbuf, obuf, lsem, ssem):
    B = x_hbm.shape[0]
    flat = p_v.shape[0]
    chunk = flat // SUBCHUNKS
    nvec = chunk // LANES
    wid = lax.axis_index("s") * NUM_CORES + lax.axis_index("c")
    base = wid * flat

    nchunks = B * SUBCHUNKS

    def load(g, k):
        b, s = g // SUBCHUNKS, g % SUBCHUNKS
        return pltpu.make_async_copy(
            x_hbm.at[b, pl.ds(base + s * chunk, chunk)], xbuf.at[k], lsem.at[k])

    def store(g, k):
        b, s = g // SUBCHUNKS, g % SUBCHUNKS
        return pltpu.make_async_copy(
            obuf.at[k], o_hbm.at[b, pl.ds(base + s * chunk, chunk)], ssem.at[k])

    for k in range(NBUF):
        load(k, k).start()

    pltpu.sync_copy(p_hbm.at[pl.ds(base, flat)], p_v)

    for g in range(nchunks):
        k = g % NBUF
        off = (g % SUBCHUNKS) * chunk
        load(g, k).wait()
        if g >= NBUF:
            store(g - NBUF, k).wait()

        @plsc.parallel_loop(0, chunk, LANES, unroll=8)
        def _add(i, k=k, off=off):
            obuf[k, pl.ds(i, LANES)] = (
                xbuf[k, pl.ds(i, LANES)] + p_v[pl.ds(off + i, LANES)])

        store(g, k).start()
        if g + NBUF < nchunks:
            load(g + NBUF, k).start()

    for g in range(nchunks - NBUF, nchunks):
        store(g, g % NBUF).wait()


def kernel(x, pos_table):
    B, T, D = x.shape
    flat = T * D // NUM_WORKERS
    mesh = plsc.VectorSubcoreMesh(core_axis_name="c", subcore_axis_name="s")
    run = functools.partial(
        pl.kernel,
        mesh=mesh,
        out_type=jax.ShapeDtypeStruct((B, T * D), jnp.float32),
        scratch_types=[
            pltpu.VMEM((flat,), jnp.float32),
            pltpu.VMEM((NBUF, flat // SUBCHUNKS), jnp.float32),
            pltpu.VMEM((NBUF, flat // SUBCHUNKS), jnp.float32),
            pltpu.SemaphoreType.DMA((NBUF,)),
            pltpu.SemaphoreType.DMA((NBUF,)),
        ],
    )(_sc_body)
    out = run(x.reshape(B, T * D), pos_table.reshape(T * D))
    return out.reshape(B, T, D)


# speedup vs baseline: 2.7986x; 1.9275x over previous
import functools

import jax
import jax.numpy as jnp
from jax import lax
from jax.experimental import pallas as pl
from jax.experimental.pallas import tpu as pltpu
from jax.experimental.pallas import tpu_sc as plsc

NUM_CORES = 2
NUM_SUBCORES = 16
NUM_WORKERS = NUM_CORES * NUM_SUBCORES
LANES = 16
NBUF = 4
SUBCHUNKS = 4


def _sc_body(x_hbm, p_hbm, o_hbm, p_v, xbuf, obuf, lsem, ssem):
    B = x_hbm.shape[0]
    rows, D = p_v.shape
    chunk = rows // SUBCHUNKS
    wid = lax.axis_index("s") * NUM_CORES + lax.axis_index("c")
    base = wid * rows

    nchunks = B * SUBCHUNKS

    def load(g, k):
        b, s = g // SUBCHUNKS, g % SUBCHUNKS
        return pltpu.make_async_copy(
            x_hbm.at[b, pl.ds(base + s * chunk, chunk)], xbuf.at[k], lsem.at[k])

    def store(g, k):
        b, s = g // SUBCHUNKS, g % SUBCHUNKS
        return pltpu.make_async_copy(
            obuf.at[k], o_hbm.at[b, pl.ds(base + s * chunk, chunk)], ssem.at[k])

    for k in range(NBUF):
        load(k, k).start()

    pltpu.sync_copy(p_hbm.at[pl.ds(base, rows)], p_v)

    for g in range(nchunks):
        k = g % NBUF
        off = (g % SUBCHUNKS) * chunk
        load(g, k).wait()
        if g >= NBUF:
            store(g - NBUF, k).wait()

        @plsc.parallel_loop(0, chunk, 1, unroll=4)
        def _add(r, k=k, off=off):
            for c in range(D // LANES):
                sl = pl.ds(c * LANES, LANES)
                obuf[k, r, sl] = xbuf[k, r, sl] + p_v[off + r, sl]

        store(g, k).start()
        if g + NBUF < nchunks:
            load(g + NBUF, k).start()

    for g in range(nchunks - NBUF, nchunks):
        store(g, g % NBUF).wait()


def kernel(x, pos_table):
    B, T, D = x.shape
    rows = T // NUM_WORKERS
    chunk = rows // SUBCHUNKS
    mesh = plsc.VectorSubcoreMesh(core_axis_name="c", subcore_axis_name="s")
    run = functools.partial(
        pl.kernel,
        mesh=mesh,
        out_type=jax.ShapeDtypeStruct((B, T, D), jnp.float32),
        scratch_types=[
            pltpu.VMEM((rows, D), jnp.float32),
            pltpu.VMEM((NBUF, chunk, D), jnp.float32),
            pltpu.VMEM((NBUF, chunk, D), jnp.float32),
            pltpu.SemaphoreType.DMA((NBUF,)),
            pltpu.SemaphoreType.DMA((NBUF,)),
        ],
    )(_sc_body)
    return run(x, pos_table)


# speedup vs baseline: 3.1523x; 1.1264x over previous
import functools

import jax
import jax.numpy as jnp
from jax import lax
from jax.experimental import pallas as pl
from jax.experimental.pallas import tpu as pltpu
from jax.experimental.pallas import tpu_sc as plsc

NUM_CORES = 2
NUM_SUBCORES = 16
NUM_WORKERS = NUM_CORES * NUM_SUBCORES
LANES = 16
NBUF = 4
SUBCHUNKS = 4


def _sc_body(x_hbm, p_hbm, o_hbm, p_v, xbuf, obuf, lsem, ssem):
    B = x_hbm.shape[0]
    rows, D = p_v.shape
    chunk = rows // SUBCHUNKS
    wid = lax.axis_index("s") * NUM_CORES + lax.axis_index("c")
    base = wid * rows

    nchunks = B * SUBCHUNKS

    def load(g, k):
        b, s = g // SUBCHUNKS, g % SUBCHUNKS
        return pltpu.make_async_copy(
            x_hbm.at[b, pl.ds(base + s * chunk, chunk)], xbuf.at[k], lsem.at[k])

    def store(g, k):
        b, s = g // SUBCHUNKS, g % SUBCHUNKS
        return pltpu.make_async_copy(
            obuf.at[k], o_hbm.at[b, pl.ds(base + s * chunk, chunk)], ssem.at[k])

    for k in range(NBUF):
        load(k, k).start()

    pltpu.sync_copy(p_hbm.at[pl.ds(base, rows)], p_v)

    @pl.loop(0, nchunks, step=NBUF)
    def _ring(g0):
        for j in range(NBUF):
            g = g0 + j
            load(g, j).wait()

            @pl.when(g0 != 0)
            def _():
                store(g - NBUF, j).wait()

            off = (g % SUBCHUNKS) * chunk

            @plsc.parallel_loop(0, chunk, 1, unroll=4)
            def _add(r):
                for c in range(D // LANES):
                    sl = pl.ds(c * LANES, LANES)
                    obuf[j, r, sl] = xbuf[j, r, sl] + p_v[off + r, sl]

            store(g, j).start()

            @pl.when(g0 + NBUF < nchunks)
            def _():
                load(g + NBUF, j).start()

    for g in range(nchunks - NBUF, nchunks):
        store(g, g % NBUF).wait()


def kernel(x, pos_table):
    B, T, D = x.shape
    rows = T // NUM_WORKERS
    chunk = rows // SUBCHUNKS
    mesh = plsc.VectorSubcoreMesh(core_axis_name="c", subcore_axis_name="s")
    run = functools.partial(
        pl.kernel,
        mesh=mesh,
        out_type=jax.ShapeDtypeStruct((B, T, D), jnp.float32),
        scratch_types=[
            pltpu.VMEM((rows, D), jnp.float32),
            pltpu.VMEM((NBUF, chunk, D), jnp.float32),
            pltpu.VMEM((NBUF, chunk, D), jnp.float32),
            pltpu.SemaphoreType.DMA((NBUF,)),
            pltpu.SemaphoreType.DMA((NBUF,)),
        ],
    )(_sc_body)
    return run(x, pos_table)
